# baseline (speedup 1.0000x reference)
import functools

import jax
import jax.numpy as jnp
from jax import lax
from jax.experimental import pallas as pl
from jax.experimental.pallas import tpu as pltpu
from jax.experimental.pallas import tpu_sc as plsc

_NM = 4
_NB = 100000
_D = 64
_B = 16384
_NW = 32
_BPW = _B // _NW
_CH = 128
_NCH = _BPW // _CH
_NST = _NM * _NCH

_mesh = plsc.VectorSubcoreMesh(core_axis_name="c", subcore_axis_name="s")


@functools.partial(
    pl.kernel,
    out_type=jax.ShapeDtypeStruct((_NM, _B, 2 * _D), jnp.float32),
    mesh=_mesh,
    compiler_params=pltpu.CompilerParams(use_tc_tiling_on_sc=True),
    scratch_types=[
        pltpu.VMEM((_NCH, _CH), jnp.int32),
        pltpu.VMEM((_NCH * 16, _CH), jnp.int32),
        pltpu.VMEM((2, _CH, 2 * _D), jnp.float32),
        pltpu.VMEM((2, _CH, 2 * _D), jnp.float32),
        pltpu.VMEM((2, _CH, 2 * _D), jnp.float32),
        pltpu.SemaphoreType.DMA,
        pltpu.SemaphoreType.DMA,
        pltpu.SemaphoreType.DMA,
        pltpu.SemaphoreType.DMA,
    ],
)
def _sigmoid_boxes_sc(pidx_hbm, poff_hbm, w_hbm, W_hbm, out_hbm,
                      pidx_v, poff_v, wrow, Wrow, zZ,
                      sem_g0, sem_g1, sem_o0, sem_o1):
    wid = lax.axis_index("s") * 2 + lax.axis_index("c")
    base = wid * _BPW
    pltpu.sync_copy(pidx_hbm.at[wid], pidx_v)
    pltpu.sync_copy(poff_hbm.at[wid], poff_v)

    sems_g = (sem_g0, sem_g1)
    sems_o = (sem_o0, sem_o1)
    gather_d = {}
    out_d = {}

    def start_gather(s):
        b = s % 2
        m, j = divmod(s, _NCH)
        gather_d[s] = (
            pltpu.async_copy(w_hbm.at[m].at[pidx_v.at[j]], wrow.at[b],
                             sems_g[b]),
            pltpu.async_copy(W_hbm.at[m].at[pidx_v.at[j]], Wrow.at[b],
                             sems_g[b]),
        )

    start_gather(0)
    for s in range(_NST):
        b = s % 2
        m, j = divmod(s, _NCH)
        if s + 1 < _NST:
            start_gather(s + 1)
        for c in gather_d.pop(s):
            c.wait()
        if s >= 2:
            out_d.pop(s - 2).wait()

        @plsc.parallel_loop(0, _CH, unroll=2)
        def _row(r):
            prow = j * 16 + r // 8
            msk = poff_v[prow, pl.ds((r % 8) * 16, 16)] > 0
            for g in range(_D // 16):
                lo = pl.ds(g * 16, 16)
                hi = pl.ds(_D + g * 16, 16)
                a = jnp.exp(-jnp.where(msk, wrow[b, r, hi], wrow[b, r, lo]))
                e = jnp.exp(-jnp.where(msk, Wrow[b, r, hi], Wrow[b, r, lo]))
                ap = a + 1.0
                ep = e + 1.0
                rr = 1.0 / (ap * ep)
                zZ[b, r, pl.ds(g * 16, 16)] = ep * rr
                zZ[b, r, pl.ds(_D + g * 16, 16)] = (a + e + 1.0) * rr

        out_d[s] = pltpu.async_copy(
            zZ.at[b], out_hbm.at[m, pl.ds(base + j * _CH, _CH)], sems_o[b])

    out_d.pop(_NST - 2).wait()
    out_d.pop(_NST - 1).wait()


def kernel(box_indices, w, W):
    idx = box_indices.astype(jnp.int32).reshape(_NW, _NCH, _CH)
    pidx = idx >> 1
    par = (idx & 1).reshape(_NW, _NCH, 16, 8, 1)
    poff = jnp.broadcast_to(par, par.shape[:4] + (16,)).reshape(
        _NW, _NCH * 16, _CH)
    wp = w.reshape(_NM, _NB // 2, 2 * _D)
    Wp = W.reshape(_NM, _NB // 2, 2 * _D)
    out = _sigmoid_boxes_sc(pidx, poff, wp, Wp)
    return out.reshape(_NM, _B, 2, _D)


# speedup vs baseline: 1.2145x; 1.2145x over previous
import functools

import jax
import jax.numpy as jnp
from jax import lax
from jax.experimental import pallas as pl
from jax.experimental.pallas import tpu as pltpu
from jax.experimental.pallas import tpu_sc as plsc

_NM = 4
_NB = 100000
_D = 64
_B = 16384
_NW = 32
_BPW = _B // _NW
_CH = 32
_NCH = _BPW // _CH

_mesh = plsc.VectorSubcoreMesh(core_axis_name="c", subcore_axis_name="s")


@functools.partial(
    pl.kernel,
    out_type=jax.ShapeDtypeStruct((_NM, _B, 2 * _D), jnp.float32),
    mesh=_mesh,
    compiler_params=pltpu.CompilerParams(use_tc_tiling_on_sc=True),
    scratch_types=[
        pltpu.VMEM((_NCH, _CH), jnp.int32),
        pltpu.VMEM((2, _CH, _NM, 2 * _D), jnp.float32),
        pltpu.VMEM((2, _NM, _CH, 2 * _D), jnp.float32),
        pltpu.SemaphoreType.DMA,
        pltpu.SemaphoreType.DMA,
        pltpu.SemaphoreType.DMA,
        pltpu.SemaphoreType.DMA,
    ],
)
def _sigmoid_boxes_sc(idx_hbm, t_hbm, out_hbm,
                      idx_v, rows, zZ,
                      sem_g0, sem_g1, sem_o0, sem_o1):
    wid = lax.axis_index("s") * 2 + lax.axis_index("c")
    base = wid * _BPW
    pltpu.sync_copy(idx_hbm.at[wid], idx_v)

    sems_g = (sem_g0, sem_g1)
    sems_o = (sem_o0, sem_o1)

    def start_gather(j, b):
        pltpu.async_copy(t_hbm.at[idx_v.at[j]], rows.at[b], sems_g[b])

    def wait_gather(j, b):
        pltpu.make_async_copy(t_hbm.at[idx_v.at[j]], rows.at[b],
                              sems_g[b]).wait()

    def start_out(j, b, m):
        pltpu.async_copy(zZ.at[b, m],
                         out_hbm.at[m, pl.ds(base + j * _CH, _CH)],
                         sems_o[b])

    def wait_out(j, b, m):
        pltpu.make_async_copy(zZ.at[b, m],
                              out_hbm.at[m, pl.ds(base + j * _CH, _CH)],
                              sems_o[b]).wait()

    start_gather(0, 0)
    start_gather(1, 1)

    @pl.loop(0, _NCH, step=2)
    def _chunk(j0):
        for b in range(2):
            j = j0 + b
            wait_gather(j, b)

            @pl.when(j0 >= 2)
            def _():
                for m in range(_NM):
                    wait_out(j - 2, b, m)

            @plsc.parallel_loop(0, _CH, unroll=1)
            def _row(r):
                for m in range(_NM):
                    for g in range(_D // 16):
                        a = jnp.exp(-rows[b, r, m, pl.ds(g * 16, 16)])
                        e = jnp.exp(-rows[b, r, m, pl.ds(_D + g * 16, 16)])
                        ap = a + 1.0
                        ep = e + 1.0
                        rr = 1.0 / (ap * ep)
                        zZ[b, m, r, pl.ds(g * 16, 16)] = ep * rr
                        zZ[b, m, r, pl.ds(_D + g * 16, 16)] = \
                            (a + e + 1.0) * rr

            for m in range(_NM):
                start_out(j, b, m)

            @pl.when(j0 + 2 < _NCH)
            def _():
                start_gather(j + 2, b)

    for jj in (_NCH - 2, _NCH - 1):
        for m in range(_NM):
            wait_out(jj, jj % 2, m)


def kernel(box_indices, w, W):
    idx_all = box_indices.astype(jnp.int32).reshape(_NW, _NCH, _CH)
    table = jnp.swapaxes(jnp.concatenate((w, W), axis=2), 0, 1)
    out = _sigmoid_boxes_sc(idx_all, table)
    return out.reshape(_NM, _B, 2, _D)


# speedup vs baseline: 1.5953x; 1.3135x over previous
import functools

import jax
import jax.numpy as jnp
from jax import lax
from jax.experimental import pallas as pl
from jax.experimental.pallas import tpu as pltpu
from jax.experimental.pallas import tpu_sc as plsc

_NM = 4
_NB = 100000
_D = 64
_B = 16384
_NW = 32
_BPW = _B // _NW
_CH = 128
_NCH = _BPW // _CH
_NST = _NM * _NCH
_BL = 2048
_NBL = -(-_NB // _BL)


def _transform_tc(wv_ref, Wv_ref, out_ref):
    x = wv_ref[0]
    y = Wv_ref[0]
    z = 1.0 / (1.0 + jnp.exp(-x))
    s = 1.0 / (1.0 + jnp.exp(-y))
    bigz = z + s - s * z
    out_ref[0, :, :_D] = z.T
    out_ref[0, :, _D:] = bigz.T


@jax.jit
def _dense_zz(wv, Wv):
    return pl.pallas_call(
        _transform_tc,
        grid=(_NM, _NBL),
        in_specs=[
            pl.BlockSpec((1, _D, _BL), lambda m, c: (m, 0, c)),
            pl.BlockSpec((1, _D, _BL), lambda m, c: (m, 0, c)),
        ],
        out_specs=pl.BlockSpec((1, _BL, 2 * _D), lambda m, c: (m, c, 0)),
        out_shape=jax.ShapeDtypeStruct((_NM, _NB, 2 * _D), jnp.float32),
    )(wv, Wv)


_mesh = plsc.VectorSubcoreMesh(core_axis_name="c", subcore_axis_name="s")


@functools.partial(
    pl.kernel,
    out_type=jax.ShapeDtypeStruct((_NM, _B, 2 * _D), jnp.float32),
    mesh=_mesh,
    compiler_params=pltpu.CompilerParams(use_tc_tiling_on_sc=True),
    scratch_types=[
        pltpu.VMEM((_NCH, _CH), jnp.int32),
        pltpu.VMEM((4, _CH, 2 * _D), jnp.float32),
        pltpu.SemaphoreType.DMA,
        pltpu.SemaphoreType.DMA,
        pltpu.SemaphoreType.DMA,
        pltpu.SemaphoreType.DMA,
        pltpu.SemaphoreType.DMA,
        pltpu.SemaphoreType.DMA,
        pltpu.SemaphoreType.DMA,
        pltpu.SemaphoreType.DMA,
    ],
)
def _lookup_sc(idx_hbm, t_hbm, out_hbm, idx_v, rows,
               g0, g1, g2, g3, o0, o1, o2, o3):
    wid = lax.axis_index("s") * 2 + lax.axis_index("c")
    base = wid * _BPW
    pltpu.sync_copy(idx_hbm.at[wid], idx_v)

    sems_g = (g0, g1, g2, g3)
    sems_o = (o0, o1, o2, o3)

    def src(s):
        m, j = divmod(s, _NCH)
        return t_hbm.at[m].at[idx_v.at[j]]

    def dst(s):
        m, j = divmod(s, _NCH)
        return out_hbm.at[m, pl.ds(base + j * _CH, _CH)]

    for s in range(4):
        pltpu.async_copy(src(s), rows.at[s % 4], sems_g[s % 4])
    for s in range(_NST):
        b = s % 4
        pltpu.make_async_copy(src(s), rows.at[b], sems_g[b]).wait()
        pltpu.async_copy(rows.at[b], dst(s), sems_o[b])
        if s + 4 < _NST:
            pltpu.make_async_copy(rows.at[b], dst(s), sems_o[b]).wait()
            pltpu.async_copy(src(s + 4), rows.at[b], sems_g[b])
    for s in range(_NST - 4, _NST):
        pltpu.make_async_copy(rows.at[s % 4], dst(s), sems_o[s % 4]).wait()


def kernel(box_indices, w, W):
    idx_all = box_indices.astype(jnp.int32).reshape(_NW, _NCH, _CH)
    wv = jnp.swapaxes(w, 1, 2)
    Wv = jnp.swapaxes(W, 1, 2)
    table = _dense_zz(wv, Wv)
    out = _lookup_sc(idx_all, table)
    return out.reshape(_NM, _B, 2, _D)


# speedup vs baseline: 2.2902x; 1.4356x over previous
import functools

import jax
import jax.numpy as jnp
from jax import lax
from jax.experimental import pallas as pl
from jax.experimental.pallas import tpu as pltpu
from jax.experimental.pallas import tpu_sc as plsc

_NM = 4
_NB = 100000
_D = 64
_B = 16384
_NW = 32
_BPW = _B // _NW
_CH = 128
_NCH = _BPW // _CH
_NST = _NM * _NCH
_BL = 4096
_NBL = -(-_NB // _BL)


def _transform_tc(wv_ref, Wv_ref, out_ref):
    x = wv_ref[0]
    y = Wv_ref[0]
    z = 1.0 / (1.0 + jnp.exp(-x))
    s = 1.0 / (1.0 + jnp.exp(-y))
    zs = jnp.concatenate((z, z + s - s * z), axis=0)
    out_ref[0] = zs.T


@jax.jit
def _dense_zz(wv, Wv):
    return pl.pallas_call(
        _transform_tc,
        grid=(_NM, _NBL),
        in_specs=[
            pl.BlockSpec((1, _D, _BL), lambda m, c: (m, 0, c)),
            pl.BlockSpec((1, _D, _BL), lambda m, c: (m, 0, c)),
        ],
        out_specs=pl.BlockSpec((1, _BL, 2 * _D), lambda m, c: (m, c, 0)),
        out_shape=jax.ShapeDtypeStruct((_NM, _NB, 2 * _D), jnp.float32),
    )(wv, Wv)


_mesh = plsc.VectorSubcoreMesh(core_axis_name="c", subcore_axis_name="s")


@functools.partial(
    pl.kernel,
    out_type=jax.ShapeDtypeStruct((_NM, _B, 2 * _D), jnp.float32),
    mesh=_mesh,
    compiler_params=pltpu.CompilerParams(use_tc_tiling_on_sc=True),
    scratch_types=[
        pltpu.VMEM((_NCH, _CH), jnp.int32),
        pltpu.VMEM((4, _CH, 2 * _D), jnp.float32),
        pltpu.SemaphoreType.DMA,
        pltpu.SemaphoreType.DMA,
        pltpu.SemaphoreType.DMA,
        pltpu.SemaphoreType.DMA,
        pltpu.SemaphoreType.DMA,
        pltpu.SemaphoreType.DMA,
        pltpu.SemaphoreType.DMA,
        pltpu.SemaphoreType.DMA,
    ],
)
def _lookup_sc(idx_hbm, t_hbm, out_hbm, idx_v, rows,
               g0, g1, g2, g3, o0, o1, o2, o3):
    wid = lax.axis_index("s") * 2 + lax.axis_index("c")
    base = wid * _BPW
    pltpu.sync_copy(idx_hbm.at[wid], idx_v)

    sems_g = (g0, g1, g2, g3)
    sems_o = (o0, o1, o2, o3)

    def src(s):
        m, j = divmod(s, _NCH)
        return t_hbm.at[m].at[idx_v.at[j]]

    def dst(s):
        m, j = divmod(s, _NCH)
        return out_hbm.at[m, pl.ds(base + j * _CH, _CH)]

    for s in range(4):
        pltpu.async_copy(src(s), rows.at[s % 4], sems_g[s % 4])
    for s in range(_NST):
        b = s % 4
        pltpu.make_async_copy(src(s), rows.at[b], sems_g[b]).wait()
        pltpu.async_copy(rows.at[b], dst(s), sems_o[b])
        if s + 4 < _NST:
            pltpu.make_async_copy(rows.at[b], dst(s), sems_o[b]).wait()
            pltpu.async_copy(src(s + 4), rows.at[b], sems_g[b])
    for s in range(_NST - 4, _NST):
        pltpu.make_async_copy(rows.at[s % 4], dst(s), sems_o[s % 4]).wait()


def kernel(box_indices, w, W):
    idx_all = box_indices.astype(jnp.int32).reshape(_NW, _NCH, _CH)
    wv = jnp.swapaxes(w, 1, 2)
    Wv = jnp.swapaxes(W, 1, 2)
    table = _dense_zz(wv, Wv)
    out = _lookup_sc(idx_all, table)
    return out.reshape(_NM, _B, 2, _D)
